# baseline (speedup 1.0000x reference)
import functools

import jax
import jax.numpy as jnp
from jax import lax
from jax.experimental import pallas as pl
from jax.experimental.pallas import tpu as pltpu
from jax.experimental.pallas import tpu_sc as plsc

NUM_ENTITIES = 1000000
EMBED_DIM = 64
BATCH = 16384

NC, NS, L = 2, 16, 16
NW = NC * NS
B_PER_W = BATCH // NW
CHUNK = 64
NCHUNK = B_PER_W // CHUNK


def _tile_body(x_hbm, y_hbm, table_hbm, out_hbm, idx_v, buf, out_v, sem):
    wid = lax.axis_index("s") * NC + lax.axis_index("c")
    base = wid * B_PER_W

    pltpu.sync_copy(x_hbm.at[pl.ds(base, B_PER_W)], idx_v.at[pl.ds(0, B_PER_W)])
    pltpu.sync_copy(y_hbm.at[pl.ds(base, B_PER_W)],
                    idx_v.at[pl.ds(B_PER_W, B_PER_W)])

    def fire(c, p):
        def g(j, carry):
            row = j * 97 + c
            for l in range(L):
                pltpu.async_copy(
                    table_hbm.at[row], buf.at[p, j * L + l], sem)
                pltpu.async_copy(
                    table_hbm.at[row], buf.at[p, CHUNK + j * L + l], sem)
            return carry
        lax.fori_loop(0, CHUNK // L, g, 0)

    def drain(p):
        pltpu.make_async_copy(
            table_hbm.at[pl.ds(0, 2 * CHUNK)], buf.at[p], sem).wait()

    fire(0, 0)
    for c in range(NCHUNK):
        p = c % 2
        drain(p)
        if c + 1 < NCHUNK:
            fire(c + 1, (c + 1) % 2)

    def zero(j, carry):
        out_v[pl.ds(j * L, L)] = jnp.zeros((L,), jnp.float32)
        return carry
    lax.fori_loop(0, B_PER_W // L, zero, 0)

    pltpu.sync_copy(out_v, out_hbm.at[pl.ds(base, B_PER_W)])


@functools.partial(jax.jit, static_argnames=())
def kernel(x, y, entity_embeddings):
    mesh = plsc.VectorSubcoreMesh(core_axis_name="c", subcore_axis_name="s")
    run = pl.kernel(
        _tile_body,
        out_type=jax.ShapeDtypeStruct((BATCH,), jnp.float32),
        mesh=mesh,
        scratch_types=[
            pltpu.VMEM((2 * B_PER_W,), jnp.int32),
            pltpu.VMEM((2, 2 * CHUNK, EMBED_DIM), jnp.float32),
            pltpu.VMEM((B_PER_W,), jnp.float32),
            pltpu.SemaphoreType.DMA,
        ],
        compiler_params=pltpu.CompilerParams(needs_layout_passes=False),
    )
    return run(x.astype(jnp.int32), y.astype(jnp.int32), entity_embeddings)


# speedup vs baseline: 1.1882x; 1.1882x over previous
import functools

import jax
import jax.numpy as jnp
from jax import lax
from jax.experimental import pallas as pl
from jax.experimental.pallas import tpu as pltpu
from jax.experimental.pallas import tpu_sc as plsc

NUM_ENTITIES = 1000000
EMBED_DIM = 64
BATCH = 16384


NC, NS, L = 2, 16, 16
NW = NC * NS
SC_B = 8192
SC_B_PER_W = SC_B // NW
SC_CHUNK = 64
SC_NCHUNK = SC_B_PER_W // SC_CHUNK


def _neg_sqrt_sc(s):
    i = plsc.bitcast(s, jnp.int32)
    t = plsc.bitcast(jnp.int32(0x5F3759DF) - (i >> 1), jnp.float32)
    half_s = s * 0.5
    for _ in range(3):
        t = t * (1.5 - half_s * t * t)
    return -(s * t)


def _sc_tile_body(x_hbm, y_hbm, table_hbm, out_hbm, idx_v, buf, out_v, sem):
    wid = lax.axis_index("s") * NC + lax.axis_index("c")
    base = wid * SC_B_PER_W

    pltpu.sync_copy(x_hbm.at[pl.ds(base, SC_B_PER_W)],
                    idx_v.at[pl.ds(0, SC_B_PER_W)])
    pltpu.sync_copy(y_hbm.at[pl.ds(base, SC_B_PER_W)],
                    idx_v.at[pl.ds(SC_B_PER_W, SC_B_PER_W)])

    lane = lax.iota(jnp.int32, L)

    def fire(c, p):
        def g(j, carry):
            ivx = idx_v[pl.ds(c * SC_CHUNK + j * L, L)]
            ivy = idx_v[pl.ds(SC_B_PER_W + c * SC_CHUNK + j * L, L)]
            for l in range(L):
                pltpu.async_copy(
                    table_hbm.at[ivx[l]], buf.at[p, j * L + l], sem)
                pltpu.async_copy(
                    table_hbm.at[ivy[l]], buf.at[p, SC_CHUNK + j * L + l], sem)
            return carry
        lax.fori_loop(0, SC_CHUNK // L, g, 0)

    def drain(p):
        pltpu.make_async_copy(
            table_hbm.at[pl.ds(0, 2 * SC_CHUNK)], buf.at[p], sem).wait()

    def compute(c, p):
        def grp(j, carry):
            res = jnp.zeros((L,), jnp.float32)
            for l in range(L):
                e = j * L + l
                sq = jnp.zeros((L,), jnp.float32)
                for k in range(EMBED_DIM // L):
                    xv = buf[p, e, pl.ds(k * L, L)]
                    yv = buf[p, SC_CHUNK + e, pl.ds(k * L, L)]
                    df = xv - yv
                    sq = sq + df * df
                s = jnp.sum(sq)
                res = jnp.where(lane == l, s, res)
            out_v[pl.ds(c * SC_CHUNK + j * L, L)] = _neg_sqrt_sc(res)
            return carry
        lax.fori_loop(0, SC_CHUNK // L, grp, 0)

    fire(0, 0)
    for c in range(SC_NCHUNK):
        p = c % 2
        drain(p)
        if c + 1 < SC_NCHUNK:
            fire(c + 1, (c + 1) % 2)
        compute(c, p)

    pltpu.sync_copy(out_v, out_hbm.at[pl.ds(base, SC_B_PER_W)])


def _sc_run(x, y, table):
    mesh = plsc.VectorSubcoreMesh(core_axis_name="c", subcore_axis_name="s")
    run = pl.kernel(
        _sc_tile_body,
        out_type=jax.ShapeDtypeStruct((SC_B,), jnp.float32),
        mesh=mesh,
        scratch_types=[
            pltpu.VMEM((2 * SC_B_PER_W,), jnp.int32),
            pltpu.VMEM((2, 2 * SC_CHUNK, EMBED_DIM), jnp.float32),
            pltpu.VMEM((SC_B_PER_W,), jnp.float32),
            pltpu.SemaphoreType.DMA,
        ],
        compiler_params=pltpu.CompilerParams(needs_layout_passes=False),
    )
    return run(x, y, table)


TC_B = BATCH - SC_B
TC_BLK = 2048
TC_NBLK = TC_B // TC_BLK
TC_CH = 256
TC_NCH = TC_BLK // TC_CH


def _tc_body(xb_s, yb_s, table_hbm, out_b, buf, sem0, sem1):
    sems = [sem0, sem1]

    def fire(c):
        p = c % 2

        def g(i, carry):
            ix = xb_s[0, 0, c * TC_CH + i]
            iy = yb_s[0, 0, c * TC_CH + i]
            pltpu.make_async_copy(
                table_hbm.at[ix], buf.at[p, i], sems[p]).start()
            pltpu.make_async_copy(
                table_hbm.at[iy], buf.at[p, TC_CH + i], sems[p]).start()
            return carry
        lax.fori_loop(0, TC_CH, g, 0)

    def drain(c):
        p = c % 2
        pltpu.make_async_copy(
            table_hbm.at[pl.ds(0, 2 * TC_CH)], buf.at[p], sems[p]).wait()

    def compute(c):
        p = c % 2
        xr = buf[p, pl.ds(0, TC_CH), :]
        yr = buf[p, pl.ds(TC_CH, TC_CH), :]
        d = xr - yr
        s = jnp.sum(d * d, axis=1)
        out_b[pl.ds(c * TC_CH, TC_CH)] = -jnp.sqrt(s)

    fire(0)
    for c in range(TC_NCH):
        drain(c)
        if c + 1 < TC_NCH:
            fire(c + 1)
        compute(c)


def _tc_run(x, y, table):
    xb = x.reshape(TC_NBLK, 1, TC_BLK)
    yb = y.reshape(TC_NBLK, 1, TC_BLK)
    return pl.pallas_call(
        _tc_body,
        grid=(TC_NBLK,),
        in_specs=[
            pl.BlockSpec((1, 1, TC_BLK), lambda i: (i, 0, 0),
                         memory_space=pltpu.SMEM),
            pl.BlockSpec((1, 1, TC_BLK), lambda i: (i, 0, 0),
                         memory_space=pltpu.SMEM),
            pl.BlockSpec(memory_space=pltpu.HBM),
        ],
        out_specs=pl.BlockSpec((TC_BLK,), lambda i: (i,)),
        out_shape=jax.ShapeDtypeStruct((TC_B,), jnp.float32),
        scratch_shapes=[
            pltpu.VMEM((2, 2 * TC_CH, EMBED_DIM), jnp.float32),
            pltpu.SemaphoreType.DMA,
            pltpu.SemaphoreType.DMA,
        ],
    )(xb, yb, table)


@functools.partial(jax.jit, static_argnames=())
def kernel(x, y, entity_embeddings):
    x = x.astype(jnp.int32)
    y = y.astype(jnp.int32)
    sc_out = _sc_run(x[:SC_B], y[:SC_B], entity_embeddings)
    tc_out = _tc_run(x[SC_B:], y[SC_B:], entity_embeddings)
    return jnp.concatenate([sc_out, tc_out])
